# baseline (speedup 1.0000x reference)
import functools

import jax
import jax.numpy as jnp
from jax import lax
from jax.experimental import pallas as pl
from jax.experimental.pallas import tpu as pltpu
from jax.experimental.pallas import tpu_sc as plsc

_RATE = 0.5
_NUM_CORES = 2
_NUM_SUBCORES = 16
_NUM_WORKERS = _NUM_CORES * _NUM_SUBCORES
_CHUNK = 8
_NBUF = 6
_LAG = 3
_TC_FRAC = 4
_TC_BLOCK = 256


def _sc_gather(x2, src_rows, sc_rows, sc_base):
    D = x2.shape[1]
    rows_per_w = sc_rows // _NUM_WORKERS
    n_iter = rows_per_w // _CHUNK

    mesh = plsc.VectorSubcoreMesh(core_axis_name="c", subcore_axis_name="s")

    @functools.partial(
        pl.kernel,
        out_type=jax.ShapeDtypeStruct((sc_rows, D), jnp.float32),
        mesh=mesh,
        scratch_types=(
            [pltpu.VMEM((rows_per_w,), jnp.int32)]
            + [pltpu.VMEM((_CHUNK, D), jnp.float32) for _ in range(_NBUF)]
            + [pltpu.SemaphoreType.DMA for _ in range(2 * _NBUF)]
        ),
    )
    def run(x_hbm, idx_hbm, o_hbm, idx_v, *scratch):
        bufs = scratch[:_NBUF]
        in_sems = scratch[_NBUF : 2 * _NBUF]
        out_sems = scratch[2 * _NBUF :]
        wid = lax.axis_index("s") * _NUM_CORES + lax.axis_index("c")
        base = wid * rows_per_w
        pltpu.sync_copy(idx_hbm.at[pl.ds(sc_base + base, rows_per_w)], idx_v)

        def start_in(i):
            return pltpu.async_copy(
                x_hbm.at[idx_v.at[pl.ds(i * _CHUNK, _CHUNK)]],
                bufs[i % _NBUF],
                in_sems[i % _NBUF],
            )

        def start_out(i):
            return pltpu.async_copy(
                bufs[i % _NBUF],
                o_hbm.at[pl.ds(base + i * _CHUNK, _CHUNK)],
                out_sems[i % _NBUF],
            )

        h_in = {}
        h_out = {}
        waited = set()
        for j in range(min(_NBUF, n_iter)):
            h_in[j] = start_in(j)
        for i in range(n_iter):
            h_in[i].wait()
            h_out[i] = start_out(i)
            j = i - _LAG
            if j >= 0 and j + _NBUF < n_iter:
                h_out[j].wait()
                waited.add(j)
                h_in[j + _NBUF] = start_in(j + _NBUF)
        for i in range(n_iter):
            if i not in waited:
                h_out[i].wait()

    return run(x2, src_rows)


def _tc_body(x_ref, o_ref):
    o_ref[...] = x_ref[::2, :]


def _tc_gather(x2, tc_rows):
    D = x2.shape[1]
    return pl.pallas_call(
        _tc_body,
        grid=(tc_rows // _TC_BLOCK, D // 128),
        in_specs=[pl.BlockSpec((2 * _TC_BLOCK, 128), lambda j, k: (j, k))],
        out_specs=pl.BlockSpec((_TC_BLOCK, 128), lambda j, k: (j, k)),
        out_shape=jax.ShapeDtypeStruct((tc_rows, D), jnp.float32),
    )(x2[: 2 * tc_rows])


def kernel(x):
    B, old_len, D = x.shape
    new_len = int(old_len * _RATE)
    R = B * new_len
    x2 = x.reshape(B * old_len, D)
    src_rows = jnp.arange(R, dtype=jnp.int32) * 2

    tc_rows = R // _TC_FRAC
    sc_rows = R - tc_rows

    out_tc = _tc_gather(x2, tc_rows)
    out_sc = _sc_gather(x2, src_rows, sc_rows, tc_rows)
    out = jnp.concatenate([out_tc, out_sc], axis=0)
    return out.reshape(B, new_len, D)


# speedup vs baseline: 2.9813x; 2.9813x over previous
import functools

import jax
import jax.numpy as jnp
from jax import lax
from jax.experimental import pallas as pl
from jax.experimental.pallas import tpu as pltpu
from jax.experimental.pallas import tpu_sc as plsc

_RATE = 0.5
_NUM_CORES = 2
_NUM_SUBCORES = 16
_NUM_WORKERS = _NUM_CORES * _NUM_SUBCORES
_CHUNK = 8
_NBUF = 6
_LAG = 3


def kernel(x):
    B, old_len, D = x.shape
    new_len = int(old_len * _RATE)
    R = B * new_len
    x2 = x.reshape(B * old_len, D)
    src_rows = jnp.arange(R, dtype=jnp.int32) * 2

    rows_per_w = R // _NUM_WORKERS
    n_iter = rows_per_w // _CHUNK

    mesh = plsc.VectorSubcoreMesh(core_axis_name="c", subcore_axis_name="s")

    @functools.partial(
        pl.kernel,
        out_type=jax.ShapeDtypeStruct((R, D), jnp.float32),
        mesh=mesh,
        scratch_types=(
            [pltpu.VMEM((rows_per_w,), jnp.int32)]
            + [pltpu.VMEM((_CHUNK, D), jnp.float32) for _ in range(_NBUF)]
            + [pltpu.SemaphoreType.DMA for _ in range(2 * _NBUF)]
        ),
    )
    def run(x_hbm, idx_hbm, o_hbm, idx_v, *scratch):
        bufs = scratch[:_NBUF]
        in_sems = scratch[_NBUF : 2 * _NBUF]
        out_sems = scratch[2 * _NBUF :]
        wid = lax.axis_index("s") * _NUM_CORES + lax.axis_index("c")
        base = wid * rows_per_w
        pltpu.sync_copy(idx_hbm.at[pl.ds(base, rows_per_w)], idx_v)

        def start_in(i):
            return pltpu.async_copy(
                x_hbm.at[idx_v.at[pl.ds(i * _CHUNK, _CHUNK)]],
                bufs[i % _NBUF],
                in_sems[i % _NBUF],
            )

        def start_out(i):
            return pltpu.async_copy(
                bufs[i % _NBUF],
                o_hbm.at[pl.ds(base + i * _CHUNK, _CHUNK)],
                out_sems[i % _NBUF],
            )

        h_in = {}
        h_out = {}
        waited = set()
        for j in range(min(_NBUF, n_iter)):
            h_in[j] = start_in(j)
        for i in range(n_iter):
            h_in[i].wait()
            h_out[i] = start_out(i)
            j = i - _LAG
            if j >= 0 and j + _NBUF < n_iter:
                h_out[j].wait()
                waited.add(j)
                h_in[j + _NBUF] = start_in(j + _NBUF)
        for i in range(n_iter):
            if i not in waited:
                h_out[i].wait()

    out = run(x2, src_rows)
    return out.reshape(B, new_len, D)


# speedup vs baseline: 2.9932x; 1.0040x over previous
import functools

import jax
import jax.numpy as jnp
from jax import lax
from jax.experimental import pallas as pl
from jax.experimental.pallas import tpu as pltpu
from jax.experimental.pallas import tpu_sc as plsc

_RATE = 0.5
_NUM_CORES = 2
_NUM_SUBCORES = 16
_NUM_WORKERS = _NUM_CORES * _NUM_SUBCORES
_LANES = 16
_CHUNK = 8
_NBUF = 6
_LAG = 3


def kernel(x):
    B, old_len, D = x.shape
    new_len = int(old_len * _RATE)
    R = B * new_len
    x2 = x.reshape(B * old_len, D)

    rows_per_w = R // _NUM_WORKERS
    n_iter = rows_per_w // _CHUNK

    mesh = plsc.VectorSubcoreMesh(core_axis_name="c", subcore_axis_name="s")

    @functools.partial(
        pl.kernel,
        out_type=jax.ShapeDtypeStruct((R, D), jnp.float32),
        mesh=mesh,
        scratch_types=(
            [pltpu.VMEM((rows_per_w,), jnp.int32)]
            + [pltpu.VMEM((_CHUNK, D), jnp.float32) for _ in range(_NBUF)]
            + [pltpu.SemaphoreType.DMA for _ in range(2 * _NBUF)]
        ),
    )
    def run(x_hbm, o_hbm, idx_v, *scratch):
        bufs = scratch[:_NBUF]
        in_sems = scratch[_NBUF : 2 * _NBUF]
        out_sems = scratch[2 * _NBUF :]
        wid = lax.axis_index("s") * _NUM_CORES + lax.axis_index("c")
        base = wid * rows_per_w
        ramp = lax.iota(jnp.int32, _LANES) * 2
        for k in range(rows_per_w // _LANES):
            idx_v[pl.ds(k * _LANES, _LANES)] = ramp + 2 * (base + k * _LANES)

        def start_in(i):
            return pltpu.async_copy(
                x_hbm.at[idx_v.at[pl.ds(i * _CHUNK, _CHUNK)]],
                bufs[i % _NBUF],
                in_sems[i % _NBUF],
            )

        def start_out(i):
            return pltpu.async_copy(
                bufs[i % _NBUF],
                o_hbm.at[pl.ds(base + i * _CHUNK, _CHUNK)],
                out_sems[i % _NBUF],
            )

        h_in = {}
        h_out = {}
        waited = set()
        for j in range(min(_NBUF, n_iter)):
            h_in[j] = start_in(j)
        for i in range(n_iter):
            h_in[i].wait()
            h_out[i] = start_out(i)
            j = i - _LAG
            if j >= 0 and j + _NBUF < n_iter:
                h_out[j].wait()
                waited.add(j)
                h_in[j + _NBUF] = start_in(j + _NBUF)
        for i in range(n_iter):
            if i not in waited:
                h_out[i].wait()

    out = run(x2)
    return out.reshape(B, new_len, D)
